# Initial kernel scaffold; baseline (speedup 1.0000x reference)
import jax
import jax.numpy as jnp
from jax.experimental import pallas as pl


def kernel(inputs):
    raise NotImplementedError("write your pallas kernel here")


# speedup vs baseline: 7.5878x; 7.5878x over previous
import functools

import numpy as np
import jax
import jax.numpy as jnp
from jax import lax
from jax.experimental import pallas as pl
from jax.experimental.pallas import tpu as pltpu
from jax.experimental.pallas import tpu_sc as plsc

B = 16
L = 2048
NSWAPS = 16
NSLOT = 2 * NSWAPS
SLOT_PAD = 128

_RATE_THRESH = 838861

_KSEL = (0x6D3E048F, 0x1022172D)

_STEP_KEYS = np.array([
    [2350016172, 1168365246], [2853785955, 313133857],
    [1914800406, 1741898942], [1770995085, 4163859872],
    [3754419596, 3661185138], [3204329507, 4228560771],
    [3913626572, 2520847663], [693197797, 1495815361],
    [1520857337, 4010142990], [1324738421, 1302467950],
    [3365621314, 850623499], [1072820213, 609676677],
    [1929251373, 1353429927], [3071341656, 3338261538],
    [2142986927, 3449929434], [3353151872, 1789448280],
], dtype=np.uint32)

_R0 = (13, 15, 26, 6)
_R1 = (17, 29, 16, 24)


def _threefry2x32(k0, k1, x0, x1):
    ks2 = k0 ^ k1 ^ jnp.int32(0x1BD11BDA)
    ks = (k0, k1, ks2)
    x0 = x0 + k0
    x1 = x1 + k1
    for g in range(5):
        for r in (_R0 if g % 2 == 0 else _R1):
            x0 = x0 + x1
            x1 = lax.shift_left(x1, jnp.int32(r)) | lax.shift_right_logical(
                x1, jnp.int32(32 - r))
            x1 = x1 ^ x0
        x0 = x0 + ks[(g + 1) % 3]
        x1 = x1 + ks[(g + 2) % 3] + jnp.int32(g + 1)
    return x0, x1


def _swap_program_kernel(k0_ref, k1_ref, pos_ref, src_ref):
    row = lax.broadcasted_iota(jnp.int32, (B, L), 0)
    col = lax.broadcasted_iota(jnp.int32, (B, L), 1)
    lin = row * jnp.int32(L) + col
    b0, b1 = _threefry2x32(jnp.int32(np.uint32(_KSEL[0]).astype(np.int32)),
                           jnp.int32(np.uint32(_KSEL[1]).astype(np.int32)),
                           jnp.zeros((B, L), jnp.int32), lin)
    bits = b0 ^ b1
    bern = (lax.shift_right_logical(bits, jnp.int32(9))
            < jnp.int32(_RATE_THRESH)).astype(jnp.int32)
    num = jnp.minimum(jnp.sum(bern, axis=1, keepdims=True), jnp.int32(NSWAPS))

    srow = lax.broadcasted_iota(jnp.int32, (B, SLOT_PAD), 0)
    slot = lax.broadcasted_iota(jnp.int32, (B, SLOT_PAD), 1)
    cnt = srow * 2 + (slot & 1)
    h0, h1 = _threefry2x32(k0_ref[...], k1_ref[...],
                           jnp.zeros((B, SLOT_PAD), jnp.int32), cnt)
    posbits = (h0 ^ h1) & jnp.int32(L - 1)
    active = ((lax.shift_right_logical(slot, jnp.int32(1)) < num)
              & (slot < jnp.int32(NSLOT)))
    pos = jnp.where(active, posbits, jnp.int32(0))

    cur = pos
    for s in range(NSWAPS):
        p1 = pos[:, 2 * s:2 * s + 1]
        p2 = pos[:, 2 * s + 1:2 * s + 2]
        a = cur[:, 2 * s:2 * s + 1]
        b = cur[:, 2 * s + 1:2 * s + 2]
        cur = jnp.where(pos == p1, b, jnp.where(pos == p2, a, cur))
    pos_ref[...] = pos
    src_ref[...] = cur


def _step_key_tables():
    k0 = np.zeros((1, SLOT_PAD), np.uint32)
    k1 = np.zeros((1, SLOT_PAD), np.uint32)
    k0[0, 0:NSLOT:2] = k0[0, 1:NSLOT:2] = _STEP_KEYS[:, 0]
    k1[0, 0:NSLOT:2] = k1[0, 1:NSLOT:2] = _STEP_KEYS[:, 1]
    bcast = lambda a: jnp.broadcast_to(
        jnp.asarray(a.astype(np.int32)), (B, SLOT_PAD))
    return bcast(k0), bcast(k1)


def _swap_program():
    k0a, k1a = _step_key_tables()
    return pl.pallas_call(
        _swap_program_kernel,
        out_shape=[jax.ShapeDtypeStruct((B, SLOT_PAD), jnp.int32)] * 2,
    )(k0a, k1a)


def _sc_apply_body(tokens_hbm, pos_hbm, src_hbm, out_hbm, rowbuf, pos_v, src_v):
    wid = lax.axis_index("s") * 2 + lax.axis_index("c")

    @pl.when(wid < B)
    def _():
        pltpu.sync_copy(tokens_hbm.at[wid], rowbuf)
        pltpu.sync_copy(pos_hbm.at[wid], pos_v)
        pltpu.sync_copy(src_hbm.at[wid], src_v)
        s0 = src_v[pl.ds(0, 16)]
        s1 = src_v[pl.ds(16, 16)]
        v0 = plsc.load_gather(rowbuf, [s0])
        v1 = plsc.load_gather(rowbuf, [s1])
        p0 = pos_v[pl.ds(0, 16)]
        p1 = pos_v[pl.ds(16, 16)]
        plsc.store_scatter(rowbuf, [p0], v0)
        plsc.store_scatter(rowbuf, [p1], v1)
        pltpu.sync_copy(rowbuf, out_hbm.at[wid])


@functools.lru_cache(maxsize=1)
def _sc_apply():
    mesh = plsc.VectorSubcoreMesh(core_axis_name="c", subcore_axis_name="s")
    return pl.kernel(
        _sc_apply_body,
        out_type=jax.ShapeDtypeStruct((B, L), jnp.int32),
        mesh=mesh,
        scratch_types=[
            pltpu.VMEM((L,), jnp.int32),
            pltpu.VMEM((SLOT_PAD,), jnp.int32),
            pltpu.VMEM((SLOT_PAD,), jnp.int32),
        ],
        compiler_params=pltpu.CompilerParams(needs_layout_passes=False),
    )


def kernel(inputs):
    tokens = inputs
    pos, src = _swap_program()
    return _sc_apply()(tokens, pos, src)


# speedup vs baseline: 8.2232x; 1.0837x over previous
import functools

import numpy as np
import jax
import jax.numpy as jnp
from jax import lax
from jax.experimental import pallas as pl
from jax.experimental.pallas import tpu as pltpu
from jax.experimental.pallas import tpu_sc as plsc

B = 16
L = 2048
NSWAPS = 16
NLANE = 16
NCHUNK = L // NLANE

_RATE_THRESH = 838861

_KSEL = (0x6D3E048F, 0x1022172D)

_STEP_KEYS = np.array([
    [2350016172, 1168365246], [2853785955, 313133857],
    [1914800406, 1741898942], [1770995085, 4163859872],
    [3754419596, 3661185138], [3204329507, 4228560771],
    [3913626572, 2520847663], [693197797, 1495815361],
    [1520857337, 4010142990], [1324738421, 1302467950],
    [3365621314, 850623499], [1072820213, 609676677],
    [1929251373, 1353429927], [3071341656, 3338261538],
    [2142986927, 3449929434], [3353151872, 1789448280],
], dtype=np.uint32)

_R0 = (13, 15, 26, 6)
_R1 = (17, 29, 16, 24)


def _threefry2x32(k0, k1, x0, x1):
    ks2 = k0 ^ k1 ^ jnp.int32(0x1BD11BDA)
    ks = (k0, k1, ks2)
    x0 = x0 + k0
    x1 = x1 + k1
    for g in range(5):
        for r in (_R0 if g % 2 == 0 else _R1):
            x0 = x0 + x1
            x1 = lax.shift_left(x1, jnp.int32(r)) | lax.shift_right_logical(
                x1, jnp.int32(32 - r))
            x1 = x1 ^ x0
        x0 = x0 + ks[(g + 1) % 3]
        x1 = x1 + ks[(g + 2) % 3] + jnp.int32(g + 1)
    return x0, x1


def _bits32(k0, k1, lin):
    b0, b1 = _threefry2x32(k0, k1, jnp.zeros_like(lin), lin)
    return b0 ^ b1


def _sc_body(tokens_hbm, keys_hbm, out_hbm, rowbuf, keybuf, sem):
    wid = lax.axis_index("s") * 2 + lax.axis_index("c")

    @pl.when(wid < B)
    def _():
        row = wid
        copy = pltpu.make_async_copy(tokens_hbm.at[row], rowbuf, sem)
        copy.start()
        pltpu.sync_copy(keys_hbm, keybuf)

        lanes = lax.iota(jnp.int32, NLANE)
        ksel0 = jnp.int32(np.uint32(_KSEL[0]).astype(np.int32))
        ksel1 = jnp.int32(np.uint32(_KSEL[1]).astype(np.int32))

        def cond(carry):
            cnt, chunk = carry
            return (cnt < NSWAPS) & (chunk < NCHUNK)

        def body(carry):
            cnt, chunk = carry
            lin = row * jnp.int32(L) + chunk * jnp.int32(NLANE) + lanes
            bits = _bits32(ksel0, ksel1, lin)
            bern = (lax.shift_right_logical(bits, jnp.int32(9))
                    < jnp.int32(_RATE_THRESH)).astype(jnp.int32)
            return cnt + jnp.sum(bern), chunk + jnp.int32(1)

        cnt, _ = lax.while_loop(cond, body, (jnp.int32(0), jnp.int32(0)))
        num = jnp.minimum(cnt, jnp.int32(NSWAPS))

        k0v = keybuf[0, :]
        k1v = keybuf[1, :]
        i1 = _bits32(k0v, k1v, jnp.zeros((NLANE,), jnp.int32)
                     + jnp.int32(2) * row) & jnp.int32(L - 1)
        i2 = _bits32(k0v, k1v, jnp.zeros((NLANE,), jnp.int32)
                     + jnp.int32(2) * row + jnp.int32(1)) & jnp.int32(L - 1)
        active = lanes < num
        pos0 = jnp.where(active, i1, jnp.int32(0))
        pos1 = jnp.where(active, i2, jnp.int32(0))

        def pick(v, lane):
            return jnp.sum(jnp.where(lanes == jnp.int32(lane), v, jnp.int32(0)))

        cur0, cur1 = pos0, pos1
        for s in range(NSWAPS):
            p1 = pick(pos0, s)
            p2 = pick(pos1, s)
            a = pick(cur0, s)
            b = pick(cur1, s)
            cur0 = jnp.where(pos0 == p1, b, jnp.where(pos0 == p2, a, cur0))
            cur1 = jnp.where(pos1 == p1, b, jnp.where(pos1 == p2, a, cur1))

        copy.wait()

        v0 = plsc.load_gather(rowbuf, [cur0])
        v1v = plsc.load_gather(rowbuf, [cur1])
        plsc.store_scatter(rowbuf, [pos0], v0)
        plsc.store_scatter(rowbuf, [pos1], v1v)

        pltpu.sync_copy(rowbuf, out_hbm.at[row])


@functools.lru_cache(maxsize=1)
def _sc_kernel():
    mesh = plsc.VectorSubcoreMesh(core_axis_name="c", subcore_axis_name="s")
    return pl.kernel(
        _sc_body,
        out_type=jax.ShapeDtypeStruct((B, L), jnp.int32),
        mesh=mesh,
        scratch_types=[
            pltpu.VMEM((L,), jnp.int32),
            pltpu.VMEM((2, NLANE), jnp.int32),
            pltpu.SemaphoreType.DMA,
        ],
        compiler_params=pltpu.CompilerParams(needs_layout_passes=False),
    )


def _step_key_table():
    return jnp.asarray(
        np.stack([_STEP_KEYS[:, 0], _STEP_KEYS[:, 1]]).astype(np.int32))


def kernel(inputs):
    tokens = inputs
    return _sc_kernel()(tokens, _step_key_table())


# speedup vs baseline: 8.7021x; 1.0582x over previous
import functools

import numpy as np
import jax
import jax.numpy as jnp
from jax import lax
from jax.experimental import pallas as pl
from jax.experimental.pallas import tpu as pltpu
from jax.experimental.pallas import tpu_sc as plsc

B = 16
L = 2048
NSWAPS = 16
NLANE = 16
NCHUNK = L // NLANE

_RATE_THRESH = 838861

_KSEL = (0x6D3E048F, 0x1022172D)

_STEP_KEYS = np.array([
    [2350016172, 1168365246], [2853785955, 313133857],
    [1914800406, 1741898942], [1770995085, 4163859872],
    [3754419596, 3661185138], [3204329507, 4228560771],
    [3913626572, 2520847663], [693197797, 1495815361],
    [1520857337, 4010142990], [1324738421, 1302467950],
    [3365621314, 850623499], [1072820213, 609676677],
    [1929251373, 1353429927], [3071341656, 3338261538],
    [2142986927, 3449929434], [3353151872, 1789448280],
], dtype=np.uint32)

_R0 = (13, 15, 26, 6)
_R1 = (17, 29, 16, 24)


def _threefry2x32(k0, k1, x0, x1):
    ks2 = k0 ^ k1 ^ jnp.int32(0x1BD11BDA)
    ks = (k0, k1, ks2)
    x0 = x0 + k0
    x1 = x1 + k1
    for g in range(5):
        for r in (_R0 if g % 2 == 0 else _R1):
            x0 = x0 + x1
            x1 = lax.shift_left(x1, jnp.int32(r)) | lax.shift_right_logical(
                x1, jnp.int32(32 - r))
            x1 = x1 ^ x0
        x0 = x0 + ks[(g + 1) % 3]
        x1 = x1 + ks[(g + 2) % 3] + jnp.int32(g + 1)
    return x0, x1


def _bits32(k0, k1, lin):
    b0, b1 = _threefry2x32(k0, k1, jnp.zeros_like(lin), lin)
    return b0 ^ b1


def _sc_body(tokens_hbm, keys_hbm, out_hbm, rowbuf, keybuf, sem):
    wid = lax.axis_index("s")

    @pl.when(wid < B)
    def _():
        row = wid
        copy = pltpu.make_async_copy(tokens_hbm.at[row], rowbuf, sem)
        copy.start()
        pltpu.sync_copy(keys_hbm, keybuf)

        lanes = lax.iota(jnp.int32, NLANE)
        ksel0 = jnp.int32(np.uint32(_KSEL[0]).astype(np.int32))
        ksel1 = jnp.int32(np.uint32(_KSEL[1]).astype(np.int32))

        def cond(carry):
            cnt, chunk = carry
            return (cnt < NSWAPS) & (chunk < NCHUNK)

        def body(carry):
            cnt, chunk = carry
            lin = row * jnp.int32(L) + chunk * jnp.int32(NLANE) + lanes
            bits = _bits32(ksel0, ksel1, lin)
            bern = (lax.shift_right_logical(bits, jnp.int32(9))
                    < jnp.int32(_RATE_THRESH)).astype(jnp.int32)
            return cnt + jnp.sum(bern), chunk + jnp.int32(1)

        cnt, _ = lax.while_loop(cond, body, (jnp.int32(0), jnp.int32(0)))
        num = jnp.minimum(cnt, jnp.int32(NSWAPS))

        k0v = keybuf[0, :]
        k1v = keybuf[1, :]
        i1 = _bits32(k0v, k1v, jnp.zeros((NLANE,), jnp.int32)
                     + jnp.int32(2) * row) & jnp.int32(L - 1)
        i2 = _bits32(k0v, k1v, jnp.zeros((NLANE,), jnp.int32)
                     + jnp.int32(2) * row + jnp.int32(1)) & jnp.int32(L - 1)
        active = lanes < num
        pos0 = jnp.where(active, i1, jnp.int32(0))
        pos1 = jnp.where(active, i2, jnp.int32(0))

        def pick(v, lane):
            return jnp.sum(jnp.where(lanes == jnp.int32(lane), v, jnp.int32(0)))

        cur0, cur1 = pos0, pos1
        for s in range(NSWAPS):
            p1 = pick(pos0, s)
            p2 = pick(pos1, s)
            a = pick(cur0, s)
            b = pick(cur1, s)
            cur0 = jnp.where(pos0 == p1, b, jnp.where(pos0 == p2, a, cur0))
            cur1 = jnp.where(pos1 == p1, b, jnp.where(pos1 == p2, a, cur1))

        copy.wait()

        v0 = plsc.load_gather(rowbuf, [cur0])
        v1v = plsc.load_gather(rowbuf, [cur1])
        plsc.store_scatter(rowbuf, [pos0], v0)
        plsc.store_scatter(rowbuf, [pos1], v1v)

        pltpu.sync_copy(rowbuf, out_hbm.at[row])


@functools.lru_cache(maxsize=1)
def _sc_kernel():
    mesh = plsc.VectorSubcoreMesh(core_axis_name="c", subcore_axis_name="s",
                                  num_cores=1)
    return pl.kernel(
        _sc_body,
        out_type=jax.ShapeDtypeStruct((B, L), jnp.int32),
        mesh=mesh,
        scratch_types=[
            pltpu.VMEM((L,), jnp.int32),
            pltpu.VMEM((2, NLANE), jnp.int32),
            pltpu.SemaphoreType.DMA,
        ],
        compiler_params=pltpu.CompilerParams(needs_layout_passes=False),
    )


def _step_key_table():
    return jnp.asarray(
        np.stack([_STEP_KEYS[:, 0], _STEP_KEYS[:, 1]]).astype(np.int32))


def kernel(inputs):
    tokens = inputs
    return _sc_kernel()(tokens, _step_key_table())


# speedup vs baseline: 9.2036x; 1.0576x over previous
import functools

import numpy as np
import jax
import jax.numpy as jnp
from jax import lax
from jax.experimental import pallas as pl
from jax.experimental.pallas import tpu as pltpu
from jax.experimental.pallas import tpu_sc as plsc

B = 16
L = 2048
NSWAPS = 16
NLANE = 16
NCHUNK = L // NLANE

_RATE_THRESH = 838861

_KSEL = (0x6D3E048F, 0x1022172D)

_STEP_KEYS = np.array([
    [2350016172, 1168365246], [2853785955, 313133857],
    [1914800406, 1741898942], [1770995085, 4163859872],
    [3754419596, 3661185138], [3204329507, 4228560771],
    [3913626572, 2520847663], [693197797, 1495815361],
    [1520857337, 4010142990], [1324738421, 1302467950],
    [3365621314, 850623499], [1072820213, 609676677],
    [1929251373, 1353429927], [3071341656, 3338261538],
    [2142986927, 3449929434], [3353151872, 1789448280],
], dtype=np.uint32)

_R0 = (13, 15, 26, 6)
_R1 = (17, 29, 16, 24)


def _threefry2x32(k0, k1, x0, x1):
    ks2 = k0 ^ k1 ^ jnp.int32(0x1BD11BDA)
    ks = (k0, k1, ks2)
    x0 = x0 + k0
    x1 = x1 + k1
    for g in range(5):
        for r in (_R0 if g % 2 == 0 else _R1):
            x0 = x0 + x1
            x1 = lax.shift_left(x1, jnp.int32(r)) | lax.shift_right_logical(
                x1, jnp.int32(32 - r))
            x1 = x1 ^ x0
        x0 = x0 + ks[(g + 1) % 3]
        x1 = x1 + ks[(g + 2) % 3] + jnp.int32(g + 1)
    return x0, x1


def _bits32(k0, k1, lin):
    b0, b1 = _threefry2x32(k0, k1, jnp.zeros_like(lin), lin)
    return b0 ^ b1


def _lane_consts(vals):
    lanes = lax.iota(jnp.int32, NLANE)
    out = jnp.zeros((NLANE,), jnp.int32)
    for i, v in enumerate(vals):
        out = jnp.where(lanes == jnp.int32(i),
                        jnp.int32(np.uint32(v).astype(np.int32)), out)
    return out


def _sc_body(tokens_hbm, out_hbm, rowbuf, sem):
    wid = lax.axis_index("s")

    @pl.when(wid < B)
    def _():
        row = wid
        copy = pltpu.make_async_copy(tokens_hbm.at[row], rowbuf, sem)
        copy.start()

        lanes = lax.iota(jnp.int32, NLANE)
        ksel0 = jnp.int32(np.uint32(_KSEL[0]).astype(np.int32))
        ksel1 = jnp.int32(np.uint32(_KSEL[1]).astype(np.int32))

        def cond(carry):
            cnt, chunk = carry
            return (cnt < NSWAPS) & (chunk < NCHUNK)

        def body(carry):
            cnt, chunk = carry
            lin0 = row * jnp.int32(L) + chunk * jnp.int32(NLANE) + lanes
            lin1 = lin0 + jnp.int32(NLANE)
            bits0 = _bits32(ksel0, ksel1, lin0)
            bits1 = _bits32(ksel0, ksel1, lin1)
            bern = ((lax.shift_right_logical(bits0, jnp.int32(9))
                     < jnp.int32(_RATE_THRESH)).astype(jnp.int32)
                    + (lax.shift_right_logical(bits1, jnp.int32(9))
                       < jnp.int32(_RATE_THRESH)).astype(jnp.int32))
            return cnt + jnp.sum(bern), chunk + jnp.int32(2)

        cnt, _ = lax.while_loop(cond, body, (jnp.int32(0), jnp.int32(0)))
        num = jnp.minimum(cnt, jnp.int32(NSWAPS))

        k0v = _lane_consts(_STEP_KEYS[:, 0])
        k1v = _lane_consts(_STEP_KEYS[:, 1])
        i1 = _bits32(k0v, k1v, jnp.zeros((NLANE,), jnp.int32)
                     + jnp.int32(2) * row) & jnp.int32(L - 1)
        i2 = _bits32(k0v, k1v, jnp.zeros((NLANE,), jnp.int32)
                     + jnp.int32(2) * row + jnp.int32(1)) & jnp.int32(L - 1)
        active = lanes < num
        pos0 = jnp.where(active, i1, jnp.int32(0))
        pos1 = jnp.where(active, i2, jnp.int32(0))

        dnums = lax.GatherDimensionNumbers(
            offset_dims=(), collapsed_slice_dims=(0,), start_index_map=(0,))

        def pick(v, lane):
            splat = jnp.zeros((NLANE, 1), jnp.int32) + jnp.int32(lane)
            return lax.gather(v, splat, dnums, slice_sizes=(1,),
                              mode=lax.GatherScatterMode.PROMISE_IN_BOUNDS)

        cur0, cur1 = pos0, pos1
        for s in range(NSWAPS):
            p1 = pick(pos0, s)
            p2 = pick(pos1, s)
            a = pick(cur0, s)
            b = pick(cur1, s)
            cur0 = jnp.where(pos0 == p1, b, jnp.where(pos0 == p2, a, cur0))
            cur1 = jnp.where(pos1 == p1, b, jnp.where(pos1 == p2, a, cur1))

        copy.wait()

        v0 = plsc.load_gather(rowbuf, [cur0])
        v1v = plsc.load_gather(rowbuf, [cur1])
        plsc.store_scatter(rowbuf, [pos0], v0)
        plsc.store_scatter(rowbuf, [pos1], v1v)

        pltpu.sync_copy(rowbuf, out_hbm.at[row])


@functools.lru_cache(maxsize=1)
def _sc_kernel():
    mesh = plsc.VectorSubcoreMesh(core_axis_name="c", subcore_axis_name="s",
                                  num_cores=1)
    return pl.kernel(
        _sc_body,
        out_type=jax.ShapeDtypeStruct((B, L), jnp.int32),
        mesh=mesh,
        scratch_types=[
            pltpu.VMEM((L,), jnp.int32),
            pltpu.SemaphoreType.DMA,
        ],
        compiler_params=pltpu.CompilerParams(needs_layout_passes=False),
    )


def kernel(inputs):
    tokens = inputs
    return _sc_kernel()(tokens)
